# Initial kernel scaffold; baseline (speedup 1.0000x reference)
import jax
import jax.numpy as jnp
from jax.experimental import pallas as pl


def kernel(origins, directions, density, u):
    raise NotImplementedError("write your pallas kernel here")


# speedup vs baseline: 51.2348x; 51.2348x over previous
import functools
import jax
import jax.numpy as jnp
from jax.experimental import pallas as pl
from jax.experimental.pallas import tpu as pltpu

N = 128
OUTW = 3 * N
Z0 = 2.0
DZ = 4.0 / 127.0


def _tc_body(d_ref, u_ref, o_ref, dir_ref, pts_ref, len_ref):
    d = d_ref[...] + 1e-5
    u = u_ref[...]
    s = jnp.sum(d, axis=1, keepdims=True)
    k_i = jax.lax.broadcasted_iota(jnp.int32, (N, N), 0)
    l_i = jax.lax.broadcasted_iota(jnp.int32, (N, N), 1)
    tri = (k_i <= l_i).astype(jnp.float32)
    cdf = jax.lax.dot_general(
        d, tri, (((1,), (0,)), ((), ())),
        preferred_element_type=jnp.float32,
        precision=jax.lax.Precision.HIGHEST,
    )
    t = u * s
    cnt = jnp.zeros_like(u)
    for j in range(N):
        cnt = cnt + jnp.where(cdf[:, j:j + 1] <= t, 1.0, 0.0)
    idx = jnp.minimum(cnt, 127.0)
    z = Z0 + idx * DZ
    len_ref[...] = z

    kk = jax.lax.broadcasted_iota(jnp.int32, (N, OUTW), 0)
    ll = jax.lax.broadcasted_iota(jnp.int32, (N, OUTW), 1)
    rep = (ll // 3 == kk).astype(jnp.float32)
    z384 = jax.lax.dot_general(
        z, rep, (((1,), (0,)), ((), ())),
        preferred_element_type=jnp.float32,
        precision=jax.lax.Precision.HIGHEST,
    )
    B = z.shape[0]
    c_i = jax.lax.broadcasted_iota(jnp.int32, (B, OUTW), 1) % 3
    o384 = jnp.where(c_i == 0, o_ref[:, 0:1],
                     jnp.where(c_i == 1, o_ref[:, 1:2], o_ref[:, 2:3]))
    d384 = jnp.where(c_i == 0, dir_ref[:, 0:1],
                     jnp.where(c_i == 1, dir_ref[:, 1:2], dir_ref[:, 2:3]))
    pts_ref[...] = o384 + z384 * d384


@jax.jit
def kernel(origins, directions, density, u):
    O = density.shape[0]
    B = 256
    grid = (O // B,)
    pts_flat, lens = pl.pallas_call(
        _tc_body,
        grid=grid,
        in_specs=[
            pl.BlockSpec((B, N), lambda i: (i, 0)),
            pl.BlockSpec((B, N), lambda i: (i, 0)),
            pl.BlockSpec((B, 3), lambda i: (i, 0)),
            pl.BlockSpec((B, 3), lambda i: (i, 0)),
        ],
        out_specs=[
            pl.BlockSpec((B, OUTW), lambda i: (i, 0)),
            pl.BlockSpec((B, N), lambda i: (i, 0)),
        ],
        out_shape=[
            jax.ShapeDtypeStruct((O, OUTW), jnp.float32),
            jax.ShapeDtypeStruct((O, N), jnp.float32),
        ],
    )(density, u, origins, directions)
    return pts_flat.reshape(O, N, 3), lens.reshape(O, N, 1)
